# baseline (speedup 1.0000x reference)
import functools

import jax
import jax.numpy as jnp
from jax import lax
from jax.experimental import pallas as pl
from jax.experimental.pallas import tpu as pltpu
from jax.experimental.pallas import tpu_sc as plsc

B = 64
N = 196
NP = 208
K = 49
D = 768
NBLK = NP // 16
_MESH = plsc.VectorSubcoreMesh(core_axis_name="c", subcore_axis_name="s")


@functools.partial(
    pl.kernel,
    out_type=[
        jax.ShapeDtypeStruct((B, 56, D), jnp.float32),
        jax.ShapeDtypeStruct((B * NP,), jnp.int32),
        jax.ShapeDtypeStruct((B * NP,), jnp.int32),
    ],
    mesh=_MESH,
    compiler_params=pltpu.CompilerParams(needs_layout_passes=False),
    scratch_types=[
        pltpu.VMEM((NP,), jnp.float32),
        pltpu.VMEM((NP,), jnp.int32),
        pltpu.VMEM((NP,), jnp.int32),
        pltpu.VMEM((56,), jnp.int32),
        pltpu.VMEM((56, D), jnp.float32),
        pltpu.SemaphoreType.DMA,
    ],
)
def _sc_kernel(x_hbm, noise_hbm, out_hbm, shuf_hbm, rank_hbm,
               noise_v, shuf_v, rank_v, idx_v, rows_v, sem):
    cid = lax.axis_index("c")
    sid = lax.axis_index("s")
    wid = sid * 2 + cid
    lane = lax.iota(jnp.int32, 16)

    for r in range(2):
        b = wid * 2 + r
        pltpu.sync_copy(noise_hbm.at[pl.ds(b * NP, NP)], noise_v)

        nb = [noise_v[pl.ds(16 * k, 16)] for k in range(NBLK)]
        iv = [lane + 16 * k for k in range(NBLK)]

        def step(j, cnt):
            jv = jnp.full((16,), j, jnp.int32)
            nj = plsc.load_gather(noise_v, [jv])
            new = []
            for k in range(NBLK):
                hit = (nj < nb[k]) | ((nj == nb[k]) & (jv < iv[k]))
                new.append(cnt[k] + hit.astype(jnp.int32))
            return tuple(new)

        zeros16 = jnp.zeros((16,), jnp.int32)
        rank = lax.fori_loop(0, N, step, tuple(zeros16 for _ in range(NBLK)))

        for k in range(NBLK):
            rank_v[pl.ds(16 * k, 16)] = rank[k]
            plsc.store_scatter(shuf_v, [rank[k]], iv[k])

        pltpu.sync_copy(rank_v, rank_hbm.at[pl.ds(b * NP, NP)])
        pltpu.sync_copy(shuf_v, shuf_hbm.at[pl.ds(b * NP, NP)])

        base = b * 197
        for c, off in enumerate((0, 16, 32, 40)):
            src = jnp.maximum(lane + off - 1, 0)
            g = plsc.load_gather(shuf_v, [src]) + (base + 1)
            if c == 0:
                g = jnp.where(lane == 0, base, g)
            idx_v[pl.ds(off, 16)] = g

        pltpu.async_copy(x_hbm.at[idx_v], rows_v, sem).wait()
        pltpu.sync_copy(rows_v, out_hbm.at[b])


@jax.jit
def kernel(x, noise):
    x_flat = x.reshape(B * 197, D)
    noise_pad = jnp.concatenate(
        [noise, jnp.full((B, NP - N), 2.0, jnp.float32)], axis=1).reshape(-1)
    out_pad, shuf, rank = _sc_kernel(x_flat, noise_pad)
    out = out_pad[:, :K + 1, :]
    shuf = shuf.reshape(B, NP)
    rank = rank.reshape(B, NP)
    remain_idx = shuf[:, :K]
    masked_idx = shuf[:, K:N]
    revert_idx = rank[:, :N]
    remain_padding_mask = jnp.ones((B, K + 1), dtype=jnp.float32)
    revert_padding_mask = jnp.ones((B, N + 1), dtype=jnp.float32)
    return (out, remain_idx, masked_idx, revert_idx,
            remain_padding_mask, revert_padding_mask)


# speedup vs baseline: 2.1748x; 2.1748x over previous
import functools

import jax
import jax.numpy as jnp
from jax import lax
from jax.experimental import pallas as pl
from jax.experimental.pallas import tpu as pltpu
from jax.experimental.pallas import tpu_sc as plsc

B = 64
N = 196
NP = 208
K = 49
D = 768
NBLK = NP // 16
_MESH = plsc.VectorSubcoreMesh(core_axis_name="c", subcore_axis_name="s")


@functools.partial(
    pl.kernel,
    out_type=[
        jax.ShapeDtypeStruct(((K + 1) * B, D), jnp.float32),
        jax.ShapeDtypeStruct((B * N,), jnp.int32),
        jax.ShapeDtypeStruct((B * N,), jnp.int32),
    ],
    mesh=_MESH,
    compiler_params=pltpu.CompilerParams(needs_layout_passes=False),
    scratch_types=[
        pltpu.VMEM((2 * N,), jnp.float32),
        pltpu.VMEM((2 * N,), jnp.int32),
        pltpu.VMEM((2 * N,), jnp.int32),
        pltpu.VMEM((K + 1,), jnp.int32),
        pltpu.VMEM((K + 1,), jnp.int32),
        pltpu.VMEM((K + 1, D), jnp.float32),
        pltpu.SemaphoreType.DMA,
    ],
)
def _sc_kernel(x_hbm, noise_hbm, out_hbm, shuf_hbm, rank_hbm,
               noise_v, shuf_v, rank_v, idx_v, oidx_v, rows_v, sem):
    cid = lax.axis_index("c")
    sid = lax.axis_index("s")
    wid = sid * 2 + cid
    lane = lax.iota(jnp.int32, 16)

    pltpu.sync_copy(noise_hbm.at[pl.ds(wid * 2 * N, 2 * N)], noise_v)

    for r in range(2):
        b = wid * 2 + r
        roff = r * N

        nb = [plsc.load_gather(noise_v, [lane + (roff + 16 * k)])
              for k in range(NBLK - 1)]
        last = plsc.load_gather(noise_v, [jnp.minimum(lane + (roff + 192), 2 * N - 1)])
        nb.append(jnp.where(lane < 4, last, 2.0))
        iv = [lane + 16 * k for k in range(NBLK)]

        def step(j, cnt):
            jv = jnp.full((16,), j, jnp.int32)
            nj = plsc.load_gather(noise_v, [jv + roff])
            new = []
            for k in range(NBLK):
                hit = (nj < nb[k]) | ((nj == nb[k]) & (jv < iv[k]))
                new.append(cnt[k] + hit.astype(jnp.int32))
            return tuple(new)

        zeros16 = jnp.zeros((16,), jnp.int32)
        rank = lax.fori_loop(0, N, step, tuple(zeros16 for _ in range(NBLK)))

        for k in range(NBLK):
            plsc.store_scatter(rank_v, [iv[k] + roff], rank[k],
                               mask=iv[k] < N)
            plsc.store_scatter(shuf_v, [rank[k] + roff], iv[k],
                               mask=rank[k] < N)

        for c in range(4):
            src = jnp.maximum(lane + (16 * c - 1), 0) + roff
            s = plsc.load_gather(shuf_v, [src])
            g = (s + 1) * B + b
            if c == 0:
                g = jnp.where(lane == 0, b, g)
            pos = lane + 16 * c
            m = pos < (K + 1)
            plsc.store_scatter(idx_v, [pos], g, mask=m)
            plsc.store_scatter(oidx_v, [pos], pos * B + b, mask=m)

        pltpu.async_copy(x_hbm.at[idx_v], rows_v, sem).wait()
        pltpu.async_copy(rows_v, out_hbm.at[oidx_v], sem).wait()

    pltpu.sync_copy(shuf_v, shuf_hbm.at[pl.ds(wid * 2 * N, 2 * N)])
    pltpu.sync_copy(rank_v, rank_hbm.at[pl.ds(wid * 2 * N, 2 * N)])


@jax.jit
def kernel(x, noise):
    x_t = jnp.transpose(x, (1, 0, 2)).reshape(197 * B, D)
    noise_flat = noise.reshape(B * N)
    out_t, shuf, rank = _sc_kernel(x_t, noise_flat)
    out = jnp.transpose(out_t.reshape(K + 1, B, D), (1, 0, 2))
    shuf = shuf.reshape(B, N)
    rank = rank.reshape(B, N)
    remain_idx = shuf[:, :K]
    masked_idx = shuf[:, K:]
    remain_padding_mask = jnp.ones((B, K + 1), dtype=jnp.float32)
    revert_padding_mask = jnp.ones((B, N + 1), dtype=jnp.float32)
    return (out, remain_idx, masked_idx, rank,
            remain_padding_mask, revert_padding_mask)


# speedup vs baseline: 3.6196x; 1.6643x over previous
import functools

import jax
import jax.numpy as jnp
from jax import lax
from jax.experimental import pallas as pl
from jax.experimental.pallas import tpu as pltpu
from jax.experimental.pallas import tpu_sc as plsc

B = 64
N = 196
NP = 208
K = 49
D = 768
NBLK = NP // 16
_MESH = plsc.VectorSubcoreMesh(core_axis_name="c", subcore_axis_name="s")


@functools.partial(
    pl.kernel,
    out_type=[
        jax.ShapeDtypeStruct(((K + 1) * B, D), jnp.float32),
        jax.ShapeDtypeStruct((B * N,), jnp.int32),
        jax.ShapeDtypeStruct((B * N,), jnp.int32),
    ],
    mesh=_MESH,
    compiler_params=pltpu.CompilerParams(needs_layout_passes=False),
    scratch_types=[
        pltpu.VMEM((2 * N,), jnp.float32),
        pltpu.VMEM((NP,), jnp.float32),
        pltpu.VMEM((NP,), jnp.int32),
        pltpu.VMEM((NP,), jnp.int32),
        pltpu.VMEM((2 * N,), jnp.int32),
        pltpu.VMEM((2 * N,), jnp.int32),
        pltpu.VMEM((K + 1,), jnp.int32),
        pltpu.VMEM((K + 1,), jnp.int32),
        pltpu.VMEM((K + 1,), jnp.int32),
        pltpu.VMEM((K + 1,), jnp.int32),
        pltpu.VMEM((K + 1, D), jnp.float32),
        pltpu.VMEM((K + 1, D), jnp.float32),
        pltpu.SemaphoreType.DMA,
        pltpu.SemaphoreType.DMA,
        pltpu.SemaphoreType.DMA,
        pltpu.SemaphoreType.DMA,
    ],
)
def _sc_kernel(x_hbm, noise_hbm, out_hbm, shuf_hbm, rank_hbm,
               noise_v, nf_v, nk_v, occ_v, shuf_v, rank_v,
               idx0_v, oidx0_v, idx1_v, oidx1_v, rows0_v, rows1_v,
               gsem0, gsem1, ssem0, ssem1):
    cid = lax.axis_index("c")
    sid = lax.axis_index("s")
    wid = sid * 2 + cid
    lane = lax.iota(jnp.int32, 16)
    zeros16 = jnp.zeros((16,), jnp.int32)
    ones16 = jnp.ones((16,), jnp.int32)

    pltpu.sync_copy(noise_hbm.at[pl.ds(wid * 2 * N, 2 * N)], noise_v)

    idx_bufs = ((idx0_v, oidx0_v, rows0_v, gsem0, ssem0),
                (idx1_v, oidx1_v, rows1_v, gsem1, ssem1))
    gathers = []

    for r in range(2):
        b = wid * 2 + r
        roff = r * N
        idx_v, oidx_v, rows_v, gsem, ssem = idx_bufs[r]

        for k in range(NBLK):
            src = jnp.minimum(lane + (roff + 16 * k), 2 * N - 1)
            v = plsc.load_gather(noise_v, [src])
            if k == NBLK - 1:
                v = jnp.where(lane < 4, v, 1.0)
            nf_v[pl.ds(16 * k, 16)] = v
            nk_v[pl.ds(16 * k, 16)] = plsc.bitcast(v, jnp.int32)

        nk = [nk_v[pl.ds(16 * k, 16)] for k in range(NBLK)]
        iv = [lane + 16 * k for k in range(NBLK)]

        def step(j, cnt):
            bj = plsc.load_gather(nk_v, [jnp.full((16,), j, jnp.int32)])
            return tuple(cnt[k] - ((bj - nk[k]) >> 31) for k in range(NBLK))

        rank = lax.fori_loop(0, N, step,
                             tuple(zeros16 for _ in range(NBLK)))

        for k in range(NBLK):
            occ_v[pl.ds(16 * k, 16)] = zeros16
        for k in range(NBLK):
            plsc.addupdate_scatter(occ_v, [rank[k]], ones16,
                                   mask=iv[k] < N)
        mx = zeros16
        for k in range(NBLK):
            mx = jnp.maximum(mx, occ_v[pl.ds(16 * k, 16)])
        has_tie = jnp.max(mx, axis=0) > 1

        for k in range(NBLK):
            plsc.store_scatter(rank_v, [iv[k] + roff], rank[k],
                               mask=iv[k] < N)
            plsc.store_scatter(shuf_v, [rank[k] + roff], iv[k],
                               mask=rank[k] < N)

        @pl.when(has_tie)
        def _fixup():
            nf = [nf_v[pl.ds(16 * k, 16)] for k in range(NBLK)]

            def step2(j, cnt):
                jv = jnp.full((16,), j, jnp.int32)
                nj = plsc.load_gather(nf_v, [jv])
                new = []
                for k in range(NBLK):
                    hit = (nj < nf[k]) | ((nj == nf[k]) & (jv < iv[k]))
                    new.append(cnt[k] + hit.astype(jnp.int32))
                return tuple(new)

            rank2 = lax.fori_loop(0, N, step2,
                                  tuple(zeros16 for _ in range(NBLK)))
            for k in range(NBLK):
                plsc.store_scatter(rank_v, [iv[k] + roff], rank2[k],
                                   mask=iv[k] < N)
                plsc.store_scatter(shuf_v, [rank2[k] + roff], iv[k],
                                   mask=rank2[k] < N)

        for c in range(4):
            src = jnp.maximum(lane + (16 * c - 1), 0) + roff
            s = plsc.load_gather(shuf_v, [src])
            g = (s + 1) * B + b
            if c == 0:
                g = jnp.where(lane == 0, b, g)
            pos = lane + 16 * c
            m = pos < (K + 1)
            plsc.store_scatter(idx_v, [pos], g, mask=m)
            plsc.store_scatter(oidx_v, [pos], pos * B + b, mask=m)

        gathers.append(pltpu.async_copy(x_hbm.at[idx_v], rows_v, gsem))

    scatters = []
    for r in range(2):
        idx_v, oidx_v, rows_v, gsem, ssem = idx_bufs[r]
        gathers[r].wait()
        scatters.append(pltpu.async_copy(rows_v, out_hbm.at[oidx_v], ssem))

    pltpu.sync_copy(shuf_v, shuf_hbm.at[pl.ds(wid * 2 * N, 2 * N)])
    pltpu.sync_copy(rank_v, rank_hbm.at[pl.ds(wid * 2 * N, 2 * N)])
    for s in scatters:
        s.wait()


@jax.jit
def kernel(x, noise):
    x_t = jnp.transpose(x, (1, 0, 2)).reshape(197 * B, D)
    noise_flat = noise.reshape(B * N)
    out_t, shuf, rank = _sc_kernel(x_t, noise_flat)
    out = jnp.transpose(out_t.reshape(K + 1, B, D), (1, 0, 2))
    shuf = shuf.reshape(B, N)
    rank = rank.reshape(B, N)
    remain_idx = shuf[:, :K]
    masked_idx = shuf[:, K:]
    remain_padding_mask = jnp.ones((B, K + 1), dtype=jnp.float32)
    revert_padding_mask = jnp.ones((B, N + 1), dtype=jnp.float32)
    return (out, remain_idx, masked_idx, rank,
            remain_padding_mask, revert_padding_mask)
